# baseline (speedup 1.0000x reference)
import functools

import jax
import jax.numpy as jnp
from jax.experimental import pallas as pl
from jax.experimental.pallas import tpu as pltpu


def _mse_onehot_kernel(n_total, x_ref, t_ref, out_ref, acc_ref):
    b = pl.program_id(0)
    h = pl.program_id(1)

    x = x_ref[0]
    t = t_ref[0]
    C, Hb, W = x.shape
    cidx = jax.lax.broadcasted_iota(jnp.int32, (C, Hb, W), 0)
    mask = (t[None, :, :] == cidx).astype(jnp.float32)
    d = x - mask
    d2 = (d * d).reshape(C * Hb, W)
    ones = jnp.ones((1, C * Hb), jnp.float32)
    part = jax.lax.dot_general(
        ones, d2, (((1,), (0,)), ((), ())),
        preferred_element_type=jnp.float32)

    first = jnp.logical_and(b == 0, h == 0)

    @pl.when(first)
    def _init():
        acc_ref[...] = part

    @pl.when(jnp.logical_not(first))
    def _accum():
        acc_ref[...] += part

    @pl.when(jnp.logical_and(b == pl.num_programs(0) - 1,
                             h == pl.num_programs(1) - 1))
    def _done():
        out_ref[0] = jnp.sum(acc_ref[...]) * (1.0 / n_total)


def kernel(output, target):
    B, C, H, W = output.shape
    target = target.astype(jnp.int32)

    HS = 2
    mean = pl.pallas_call(
        functools.partial(_mse_onehot_kernel, float(B * C * H * W)),
        grid=(B, HS),
        in_specs=[
            pl.BlockSpec((1, C, H // HS, W), lambda b, h: (b, 0, h, 0)),
            pl.BlockSpec((1, H // HS, W), lambda b, h: (b, h, 0)),
        ],
        out_specs=pl.BlockSpec(memory_space=pltpu.SMEM),
        out_shape=jax.ShapeDtypeStruct((1,), jnp.float32),
        scratch_shapes=[pltpu.VMEM((1, W), jnp.float32)],
        compiler_params=pltpu.CompilerParams(vmem_limit_bytes=100 * 1024 * 1024),
    )(output, target)

    return mean[0]
